# baseline (speedup 1.0000x reference)
import jax
import jax.numpy as jnp
from jax import lax
from jax.experimental import pallas as pl
from jax.experimental.pallas import tpu as pltpu
from jax.experimental.pallas import tpu_sc as plsc

_NC = 2
_NS = 16
_L = 16

_N = 10000
_E = 320000
_H = 128

_EPT = _E // (_NC * _NS)
_CH = 80
_NCHUNK = _EPT // _CH
_WB = 200
_NWB = _N // _WB
_ZC = 1000
_ZB = 40
_DK = 5
_SCK = 25
_SE = _SCK * _CH
_NSUP = _NCHUNK // _SCK


def _deg_body(dst_hbm, ew_hbm, degp_hbm, deg_s, dst_i, ew_i, zbuf_v, sem):
    c = lax.axis_index("c")
    s = lax.axis_index("s")
    wid = c * _NS + s

    def _z16(i, carry):
        zbuf_v[pl.ds(i * _L, _L)] = jnp.zeros((_L,), jnp.float32)
        return carry

    lax.fori_loop(0, _ZC // _L, _z16, 0)

    @pl.when(s < _N // _ZC)
    def _zero_acc():
        pltpu.sync_copy(zbuf_v, deg_s.at[pl.ds(s * _ZC, _ZC)])

    pltpu.sync_copy(dst_hbm.at[wid], dst_i)
    pltpu.sync_copy(ew_hbm.at[pl.ds(wid * _EPT, _EPT)], ew_i)
    plsc.subcore_barrier()

    def _grp(m, carry):
        for i in range(_DK):
            t = m * _DK + i
            pltpu.async_copy(ew_i.at[pl.ds(t * _CH, _CH)],
                             deg_s.at[dst_i.at[t]], sem, add=True)
        for i in range(_DK):
            t = m * _DK + i
            pltpu.make_async_copy(ew_i.at[pl.ds(t * _CH, _CH)],
                                  deg_s.at[dst_i.at[t]], sem).wait()
        return carry

    lax.fori_loop(0, _NCHUNK // _DK, _grp, 0)
    plsc.subcore_barrier()

    @pl.when(s == 0)
    def _writeback():
        pltpu.sync_copy(deg_s, degp_hbm.at[c])


def _agg_body(y_hbm, src_hbm, dst_hbm, ew_hbm, aggp_hbm,
              acc_s, src_i, dst_i, ew_i, rows0, rows1, zbuf_v, sem0, sem1):
    c = lax.axis_index("c")
    s = lax.axis_index("s")
    wid = c * _NS + s

    def _zrow(i, carry):
        for j in range(_H // _L):
            zbuf_v[i, pl.ds(j * _L, _L)] = jnp.zeros((_L,), jnp.float32)
        return carry

    lax.fori_loop(0, _ZB, _zrow, 0)

    for k in range((_NWB + _NS - 1) // _NS):
        blk = s + _NS * k

        @pl.when(blk < _NWB)
        def _zacc(blk=blk):
            for q in range(_WB // _ZB):
                pltpu.sync_copy(zbuf_v, acc_s.at[pl.ds(blk * _WB + q * _ZB, _ZB)])

    plsc.subcore_barrier()

    def _sup(sup, carry):
        ebase = wid * _EPT + sup * _SE
        pltpu.sync_copy(src_hbm.at[pl.ds(ebase, _SE)], src_i)
        pltpu.sync_copy(ew_hbm.at[pl.ds(ebase, _SE)], ew_i)
        pltpu.sync_copy(dst_hbm.at[wid, sup], dst_i)

        def _half(tc, rows_p, sem_p, rows_q, sem_q, ptc):
            pltpu.async_copy(y_hbm.at[src_i.at[pl.ds(ptc * _CH, _CH)]],
                             rows_q, sem_q)
            pltpu.make_async_copy(y_hbm.at[src_i.at[pl.ds(tc * _CH, _CH)]],
                                  rows_p, sem_p).wait()

            def _scale(g, inner):
                wv = ew_i[pl.ds(tc * _CH + g * _L, _L)]
                for k in range(_L):
                    w = wv[k]
                    e = g * _L + k
                    for j in range(_H // _L):
                        sl = pl.ds(j * _L, _L)
                        rows_p[e, sl] = rows_p[e, sl] * w
                return inner

            lax.fori_loop(0, _CH // _L, _scale, 0)
            pltpu.sync_copy(rows_p, acc_s.at[dst_i.at[tc]], add=True)

        pltpu.async_copy(y_hbm.at[src_i.at[pl.ds(0, _CH)]], rows0, sem0)

        def _pair(t2, carry2):
            t0 = 2 * t2
            _half(t0, rows0, sem0, rows1, sem1, t0 + 1)
            _half(t0 + 1, rows1, sem1, rows0, sem0, t0 + 2)
            return carry2

        lax.fori_loop(0, (_SCK - 1) // 2, _pair, 0)

        tl = _SCK - 1
        pltpu.make_async_copy(y_hbm.at[src_i.at[pl.ds(tl * _CH, _CH)]],
                              rows0, sem0).wait()

        def _scale_last(g, inner):
            wv = ew_i[pl.ds(tl * _CH + g * _L, _L)]
            for k in range(_L):
                w = wv[k]
                e = g * _L + k
                for j in range(_H // _L):
                    sl = pl.ds(j * _L, _L)
                    rows0[e, sl] = rows0[e, sl] * w
            return inner

        lax.fori_loop(0, _CH // _L, _scale_last, 0)
        pltpu.sync_copy(rows0, acc_s.at[dst_i.at[tl]], add=True)
        return carry

    lax.fori_loop(0, _NSUP, _sup, 0)
    plsc.subcore_barrier()

    for k in range((_NWB + _NS - 1) // _NS):
        blk = s + _NS * k

        @pl.when(blk < _NWB)
        def _wb(blk=blk):
            r0 = blk * _WB
            pltpu.sync_copy(acc_s.at[pl.ds(r0, _WB)],
                            aggp_hbm.at[c, pl.ds(r0, _WB)])


_sc_mesh = plsc.VectorSubcoreMesh(
    core_axis_name="c", subcore_axis_name="s",
    num_cores=_NC, num_subcores=_NS)

_deg_call = pl.kernel(
    _deg_body,
    out_type=jax.ShapeDtypeStruct((_NC, _N), jnp.float32),
    mesh=_sc_mesh,
    scratch_types=[
        pltpu.VMEM_SHARED((_N,), jnp.float32),
        pltpu.VMEM((_NCHUNK, _CH), jnp.int32),
        pltpu.VMEM((_EPT,), jnp.float32),
        pltpu.VMEM((_ZC,), jnp.float32),
        pltpu.SemaphoreType.DMA,
    ],
)

_agg_call = pl.kernel(
    _agg_body,
    out_type=jax.ShapeDtypeStruct((_NC, _N, _H), jnp.float32),
    mesh=_sc_mesh,
    scratch_types=[
        pltpu.VMEM_SHARED((_N, _H), jnp.float32),
        pltpu.VMEM((_SE,), jnp.int32),
        pltpu.VMEM((_SCK, _CH), jnp.int32),
        pltpu.VMEM((_SE,), jnp.float32),
        pltpu.VMEM((_CH, _H), jnp.float32),
        pltpu.VMEM((_CH, _H), jnp.float32),
        pltpu.VMEM((_ZB, _H), jnp.float32),
        pltpu.SemaphoreType.DMA,
        pltpu.SemaphoreType.DMA,
    ],
)


def _prep_tc(degp_ref, node_ref, w1_ref, b1_ref, y_ref, st_ref, dis_ref, inv_ref):
    deg = degp_ref[:, 0:1] + degp_ref[:, 1:2] + 1.0
    dis = lax.rsqrt(deg)
    inv = 1.0 / deg
    xw = jnp.dot(node_ref[...], w1_ref[...], preferred_element_type=jnp.float32)
    y_ref[...] = xw * dis
    st_ref[...] = xw * inv + b1_ref[...][None, :]
    dis_ref[...] = dis
    inv_ref[...] = inv


def _combine_tc(aggp_ref, st_ref, dis_ref, inv_ref, w_ref, b_ref, y_ref, stn_ref):
    agg = aggp_ref[0] + aggp_ref[1]
    x = jnp.maximum(dis_ref[...] * agg + st_ref[...], 0.0)
    xw = jnp.dot(x, w_ref[...], preferred_element_type=jnp.float32)
    y_ref[...] = xw * dis_ref[...]
    stn_ref[...] = xw * inv_ref[...] + b_ref[...][None, :] + x


def _final_tc(aggp_ref, st_ref, dis_ref, wfc_ref, bfc_ref, out_ref):
    agg = aggp_ref[0] + aggp_ref[1]
    x = jnp.maximum(dis_ref[...] * agg + st_ref[...], 0.0)
    m = jnp.mean(x, axis=0, keepdims=True)
    out_ref[...] = (jnp.dot(m, wfc_ref[...], preferred_element_type=jnp.float32)
                    + bfc_ref[...][None, :])


def _prep_call(degp_t, node, W1, b1):
    return pl.pallas_call(
        _prep_tc,
        out_shape=(
            jax.ShapeDtypeStruct((_N, _H), jnp.float32),
            jax.ShapeDtypeStruct((_N, _H), jnp.float32),
            jax.ShapeDtypeStruct((_N, 1), jnp.float32),
            jax.ShapeDtypeStruct((_N, 1), jnp.float32),
        ),
    )(degp_t, node, W1, b1)


def _combine_call(aggp, st, dis2, inv2, W, b):
    return pl.pallas_call(
        _combine_tc,
        out_shape=(
            jax.ShapeDtypeStruct((_N, _H), jnp.float32),
            jax.ShapeDtypeStruct((_N, _H), jnp.float32),
        ),
    )(aggp, st, dis2, inv2, W, b)


def _final_call(aggp, st, dis2, Wfc, bfc):
    return pl.pallas_call(
        _final_tc,
        out_shape=jax.ShapeDtypeStruct((1, 2), jnp.float32),
    )(aggp, st, dis2, Wfc, bfc)


def kernel(node, edges, edges_attr, W1, b1, W2_0, b2_0, W2_1, b2_1, W2_2, b2_2, Wfc, bfc):
    src, dst = edges[0], edges[1]
    order = jnp.argsort(src)
    src = src[order]
    dst = dst[order]
    edges_attr = edges_attr[order]
    nt = _NC * _NS
    dst3 = dst.reshape(nt, _NCHUNK, _CH)
    dst4 = dst.reshape(nt, _NSUP, _SCK, _CH)
    degp = _deg_call(dst3, edges_attr)
    degp_t = degp.T
    y, st, dis2, inv2 = _prep_call(degp_t, node, W1, b1)
    for (W, b) in ((W2_0, b2_0), (W2_1, b2_1), (W2_2, b2_2)):
        aggp = _agg_call(y, src, dst4, edges_attr)
        y, st = _combine_call(aggp, st, dis2, inv2, W, b)
    aggp = _agg_call(y, src, dst4, edges_attr)
    return _final_call(aggp, st, dis2, Wfc, bfc)


# speedup vs baseline: 2.5661x; 2.5661x over previous
import jax
import jax.numpy as jnp
from jax import lax
from jax.experimental import pallas as pl
from jax.experimental.pallas import tpu as pltpu
from jax.experimental.pallas import tpu_sc as plsc

_NC = 2
_NS = 16
_L = 16

_N = 10000
_E = 320000
_H = 128

_EPT = _E // (_NC * _NS)
_CH = 80
_NCHUNK = _EPT // _CH
_WB = 200
_NWB = _N // _WB
_ZC = 1000
_ZB = 40
_DK = 5
_NB = 4
_SCK = 25
_SE = _SCK * _CH
_NSUP = _NCHUNK // _SCK


def _deg_body(dst_hbm, ew_hbm, degp_hbm, deg_s, dst_i, ew_i, zbuf_v, sem):
    c = lax.axis_index("c")
    s = lax.axis_index("s")
    wid = c * _NS + s

    def _z16(i, carry):
        zbuf_v[pl.ds(i * _L, _L)] = jnp.zeros((_L,), jnp.float32)
        return carry

    lax.fori_loop(0, _ZC // _L, _z16, 0)

    @pl.when(s < _N // _ZC)
    def _zero_acc():
        pltpu.sync_copy(zbuf_v, deg_s.at[pl.ds(s * _ZC, _ZC)])

    pltpu.sync_copy(dst_hbm.at[wid], dst_i)
    pltpu.sync_copy(ew_hbm.at[pl.ds(wid * _EPT, _EPT)], ew_i)
    plsc.subcore_barrier()

    def _grp(m, carry):
        for i in range(_DK):
            t = m * _DK + i
            pltpu.async_copy(ew_i.at[pl.ds(t * _CH, _CH)],
                             deg_s.at[dst_i.at[t]], sem, add=True)
        for i in range(_DK):
            t = m * _DK + i
            pltpu.make_async_copy(ew_i.at[pl.ds(t * _CH, _CH)],
                                  deg_s.at[dst_i.at[t]], sem).wait()
        return carry

    lax.fori_loop(0, _NCHUNK // _DK, _grp, 0)
    plsc.subcore_barrier()

    @pl.when(s == 0)
    def _writeback():
        pltpu.sync_copy(deg_s, degp_hbm.at[c])


def _agg_body(y_hbm, src_hbm, dst_hbm, ew_hbm, aggp_hbm,
              acc_s, src_i, dst_i, ew_i, rows0, rows1, rows2, rows3,
              sem0, sem1, sem2, sem3):
    c = lax.axis_index("c")
    s = lax.axis_index("s")
    wid = c * _NS + s
    rows = (rows0, rows1, rows2, rows3)
    sems = (sem0, sem1, sem2, sem3)

    def _zrow(i, carry):
        for j in range(_H // _L):
            rows0[i, pl.ds(j * _L, _L)] = jnp.zeros((_L,), jnp.float32)
        return carry

    lax.fori_loop(0, _CH, _zrow, 0)

    for k in range((_NWB + _NS - 1) // _NS):
        blk = s + _NS * k

        @pl.when(blk < _NWB)
        def _zacc(blk=blk):
            r0 = blk * _WB
            pltpu.sync_copy(rows0.at[pl.ds(0, _CH)], acc_s.at[pl.ds(r0, _CH)])
            pltpu.sync_copy(rows0.at[pl.ds(0, _CH)],
                            acc_s.at[pl.ds(r0 + _CH, _CH)])
            pltpu.sync_copy(rows0.at[pl.ds(0, _WB - 2 * _CH)],
                            acc_s.at[pl.ds(r0 + 2 * _CH, _WB - 2 * _CH)])

    plsc.subcore_barrier()

    def _prefetch(t, buf):
        pltpu.async_copy(y_hbm.at[src_i.at[pl.ds(t * _CH, _CH)]],
                         rows[buf], sems[buf])

    def _process(t, buf):
        pltpu.make_async_copy(y_hbm.at[src_i.at[pl.ds(t * _CH, _CH)]],
                              rows[buf], sems[buf]).wait()
        rp = rows[buf]

        def _scale(g, inner):
            wv = ew_i[pl.ds(t * _CH + g * _L, _L)]
            for k in range(_L):
                w = wv[k]
                e = g * _L + k
                for j in range(_H // _L):
                    sl = pl.ds(j * _L, _L)
                    rp[e, sl] = rp[e, sl] * w
            return inner

        lax.fori_loop(0, _CH // _L, _scale, 0)
        pltpu.sync_copy(rp, acc_s.at[dst_i.at[t]], add=True)

    def _sup(sup, carry):
        ebase = wid * _EPT + sup * _SE
        pltpu.sync_copy(src_hbm.at[pl.ds(ebase, _SE)], src_i)
        pltpu.sync_copy(ew_hbm.at[pl.ds(ebase, _SE)], ew_i)
        pltpu.sync_copy(dst_hbm.at[wid, sup], dst_i)

        for i in range(_NB - 1):
            _prefetch(i, i)

        def _grp(m, carry2):
            t0 = m * _NB
            for i in range(_NB):
                _prefetch(t0 + i + _NB - 1, (i + _NB - 1) % _NB)
                _process(t0 + i, i)
            return carry2

        nsteady = (_SCK - (_NB - 1)) // _NB
        lax.fori_loop(0, nsteady, _grp, 0)
        for t in range(nsteady * _NB, _SCK):
            pt = t + _NB - 1
            if pt < _SCK:
                _prefetch(pt, pt % _NB)
            _process(t, t % _NB)
        return carry

    lax.fori_loop(0, _NSUP, _sup, 0)
    plsc.subcore_barrier()

    for k in range((_NWB + _NS - 1) // _NS):
        blk = s + _NS * k

        @pl.when(blk < _NWB)
        def _wb(blk=blk):
            r0 = blk * _WB
            pltpu.sync_copy(acc_s.at[pl.ds(r0, _WB)],
                            aggp_hbm.at[c, pl.ds(r0, _WB)])


_sc_mesh = plsc.VectorSubcoreMesh(
    core_axis_name="c", subcore_axis_name="s",
    num_cores=_NC, num_subcores=_NS)

_deg_call = pl.kernel(
    _deg_body,
    out_type=jax.ShapeDtypeStruct((_NC, _N), jnp.float32),
    mesh=_sc_mesh,
    scratch_types=[
        pltpu.VMEM_SHARED((_N,), jnp.float32),
        pltpu.VMEM((_NCHUNK, _CH), jnp.int32),
        pltpu.VMEM((_EPT,), jnp.float32),
        pltpu.VMEM((_ZC,), jnp.float32),
        pltpu.SemaphoreType.DMA,
    ],
)

_agg_call = pl.kernel(
    _agg_body,
    out_type=jax.ShapeDtypeStruct((_NC, _N, _H), jnp.float32),
    mesh=_sc_mesh,
    scratch_types=[
        pltpu.VMEM_SHARED((_N, _H), jnp.float32),
        pltpu.VMEM((_SE,), jnp.int32),
        pltpu.VMEM((_SCK, _CH), jnp.int32),
        pltpu.VMEM((_SE,), jnp.float32),
        pltpu.VMEM((_CH, _H), jnp.float32),
        pltpu.VMEM((_CH, _H), jnp.float32),
        pltpu.VMEM((_CH, _H), jnp.float32),
        pltpu.VMEM((_CH, _H), jnp.float32),
        pltpu.SemaphoreType.DMA,
        pltpu.SemaphoreType.DMA,
        pltpu.SemaphoreType.DMA,
        pltpu.SemaphoreType.DMA,
    ],
)


def _prep_tc(degp_ref, node_ref, w1_ref, b1_ref, y_ref, st_ref, dis_ref, inv_ref):
    deg = degp_ref[:, 0:1] + degp_ref[:, 1:2] + 1.0
    dis = lax.rsqrt(deg)
    inv = 1.0 / deg
    xw = jnp.dot(node_ref[...], w1_ref[...], preferred_element_type=jnp.float32)
    y_ref[...] = xw * dis
    st_ref[...] = xw * inv + b1_ref[...][None, :]
    dis_ref[...] = dis
    inv_ref[...] = inv


def _combine_tc(aggp_ref, st_ref, dis_ref, inv_ref, w_ref, b_ref, y_ref, stn_ref):
    agg = aggp_ref[0] + aggp_ref[1]
    x = jnp.maximum(dis_ref[...] * agg + st_ref[...], 0.0)
    xw = jnp.dot(x, w_ref[...], preferred_element_type=jnp.float32)
    y_ref[...] = xw * dis_ref[...]
    stn_ref[...] = xw * inv_ref[...] + b_ref[...][None, :] + x


def _final_tc(aggp_ref, st_ref, dis_ref, wfc_ref, bfc_ref, out_ref):
    agg = aggp_ref[0] + aggp_ref[1]
    x = jnp.maximum(dis_ref[...] * agg + st_ref[...], 0.0)
    m = jnp.mean(x, axis=0, keepdims=True)
    out_ref[...] = (jnp.dot(m, wfc_ref[...], preferred_element_type=jnp.float32)
                    + bfc_ref[...][None, :])


def _prep_call(degp_t, node, W1, b1):
    return pl.pallas_call(
        _prep_tc,
        out_shape=(
            jax.ShapeDtypeStruct((_N, _H), jnp.float32),
            jax.ShapeDtypeStruct((_N, _H), jnp.float32),
            jax.ShapeDtypeStruct((_N, 1), jnp.float32),
            jax.ShapeDtypeStruct((_N, 1), jnp.float32),
        ),
    )(degp_t, node, W1, b1)


def _combine_call(aggp, st, dis2, inv2, W, b):
    return pl.pallas_call(
        _combine_tc,
        out_shape=(
            jax.ShapeDtypeStruct((_N, _H), jnp.float32),
            jax.ShapeDtypeStruct((_N, _H), jnp.float32),
        ),
    )(aggp, st, dis2, inv2, W, b)


def _final_call(aggp, st, dis2, Wfc, bfc):
    return pl.pallas_call(
        _final_tc,
        out_shape=jax.ShapeDtypeStruct((1, 2), jnp.float32),
    )(aggp, st, dis2, Wfc, bfc)


def kernel(node, edges, edges_attr, W1, b1, W2_0, b2_0, W2_1, b2_1, W2_2, b2_2, Wfc, bfc):
    src, dst = edges[0], edges[1]
    nt = _NC * _NS
    dst3 = dst.reshape(nt, _NCHUNK, _CH)
    dst4 = dst.reshape(nt, _NSUP, _SCK, _CH)
    degp = _deg_call(dst3, edges_attr)
    degp_t = degp.T
    y, st, dis2, inv2 = _prep_call(degp_t, node, W1, b1)
    for (W, b) in ((W2_0, b2_0), (W2_1, b2_1), (W2_2, b2_2)):
        aggp = _agg_call(y, src, dst4, edges_attr)
        y, st = _combine_call(aggp, st, dis2, inv2, W, b)
    aggp = _agg_call(y, src, dst4, edges_attr)
    return _final_call(aggp, st, dis2, Wfc, bfc)


# speedup vs baseline: 2.6221x; 1.0218x over previous
import jax
import jax.numpy as jnp
from jax import lax
from jax.experimental import pallas as pl
from jax.experimental.pallas import tpu as pltpu
from jax.experimental.pallas import tpu_sc as plsc

_NC = 2
_NS = 16
_L = 16

_N = 10000
_E = 320000
_H = 128

_EPT = _E // (_NC * _NS)
_CH = 80
_NCHUNK = _EPT // _CH
_WB = 200
_NWB = _N // _WB
_ZC = 1000
_ZB = 40
_DK = 5
_NB = 4
_SCK = 25
_SE = _SCK * _CH
_NSUP = _NCHUNK // _SCK


def _deg_body(dst_hbm, ew_hbm, degp_hbm, deg_s, dst_i, ew_i, zbuf_v, sem):
    c = lax.axis_index("c")
    s = lax.axis_index("s")
    wid = c * _NS + s

    def _z16(i, carry):
        zbuf_v[pl.ds(i * _L, _L)] = jnp.zeros((_L,), jnp.float32)
        return carry

    lax.fori_loop(0, _ZC // _L, _z16, 0)

    @pl.when(s < _N // _ZC)
    def _zero_acc():
        pltpu.sync_copy(zbuf_v, deg_s.at[pl.ds(s * _ZC, _ZC)])

    pltpu.sync_copy(dst_hbm.at[wid], dst_i)
    pltpu.sync_copy(ew_hbm.at[pl.ds(wid * _EPT, _EPT)], ew_i)
    plsc.subcore_barrier()

    def _grp(m, carry):
        for i in range(_DK):
            t = m * _DK + i
            pltpu.async_copy(ew_i.at[pl.ds(t * _CH, _CH)],
                             deg_s.at[dst_i.at[t]], sem, add=True)
        for i in range(_DK):
            t = m * _DK + i
            pltpu.make_async_copy(ew_i.at[pl.ds(t * _CH, _CH)],
                                  deg_s.at[dst_i.at[t]], sem).wait()
        return carry

    lax.fori_loop(0, _NCHUNK // _DK, _grp, 0)
    plsc.subcore_barrier()

    @pl.when(s == 0)
    def _writeback():
        pltpu.sync_copy(deg_s, degp_hbm.at[c])


def _agg_body(y_hbm, src_hbm, dst_hbm, ew_hbm, aggp_hbm,
              acc_s, src_i, dst_i, ew_i, rows0, rows1, rows2, rows3,
              sem0, sem1, sem2, sem3, ssem0, ssem1, ssem2, ssem3):
    c = lax.axis_index("c")
    s = lax.axis_index("s")
    wid = c * _NS + s
    rows = (rows0, rows1, rows2, rows3)
    sems = (sem0, sem1, sem2, sem3)
    ssems = (ssem0, ssem1, ssem2, ssem3)

    def _zrow(i, carry):
        for j in range(_H // _L):
            rows0[i, pl.ds(j * _L, _L)] = jnp.zeros((_L,), jnp.float32)
        return carry

    lax.fori_loop(0, _CH, _zrow, 0)

    for k in range((_NWB + _NS - 1) // _NS):
        blk = s + _NS * k

        @pl.when(blk < _NWB)
        def _zacc(blk=blk):
            r0 = blk * _WB
            pltpu.sync_copy(rows0.at[pl.ds(0, _CH)], acc_s.at[pl.ds(r0, _CH)])
            pltpu.sync_copy(rows0.at[pl.ds(0, _CH)],
                            acc_s.at[pl.ds(r0 + _CH, _CH)])
            pltpu.sync_copy(rows0.at[pl.ds(0, _WB - 2 * _CH)],
                            acc_s.at[pl.ds(r0 + 2 * _CH, _WB - 2 * _CH)])

    plsc.subcore_barrier()

    def _prefetch(t, buf):
        pltpu.async_copy(y_hbm.at[src_i.at[pl.ds(t * _CH, _CH)]],
                         rows[buf], sems[buf])

    def _scat_wait(t, buf):
        pltpu.make_async_copy(rows[buf], acc_s.at[dst_i.at[t]],
                              ssems[buf]).wait()

    def _process(t, buf):
        pltpu.make_async_copy(y_hbm.at[src_i.at[pl.ds(t * _CH, _CH)]],
                              rows[buf], sems[buf]).wait()
        rp = rows[buf]

        def _scale(g, inner):
            wv = ew_i[pl.ds(t * _CH + g * _L, _L)]
            for k in range(_L):
                w = wv[k]
                e = g * _L + k
                for j in range(_H // _L):
                    sl = pl.ds(j * _L, _L)
                    rp[e, sl] = rp[e, sl] * w
            return inner

        lax.fori_loop(0, _CH // _L, _scale, 0)
        pltpu.async_copy(rp, acc_s.at[dst_i.at[t]], ssems[buf], add=True)

    def _drain_tail():
        for t in range(_SCK - _NB, _SCK):
            _scat_wait(t, t % _NB)

    def _sup(sup, carry):
        @pl.when(sup > 0)
        def _drain_prev():
            _drain_tail()

        ebase = wid * _EPT + sup * _SE
        pltpu.sync_copy(src_hbm.at[pl.ds(ebase, _SE)], src_i)
        pltpu.sync_copy(ew_hbm.at[pl.ds(ebase, _SE)], ew_i)
        pltpu.sync_copy(dst_hbm.at[wid, sup], dst_i)

        for i in range(_NB - 1):
            _prefetch(i, i)

        def _grp(m, carry2):
            t0 = m * _NB
            for i in range(_NB):
                tp = t0 + i + _NB - 1
                bufp = (i + _NB - 1) % _NB

                @pl.when(tp >= _NB)
                def _w(tp=tp, bufp=bufp):
                    _scat_wait(tp - _NB, bufp)

                _prefetch(tp, bufp)
                _process(t0 + i, i)
            return carry2

        nsteady = (_SCK - (_NB - 1)) // _NB
        lax.fori_loop(0, nsteady, _grp, 0)
        for t in range(nsteady * _NB, _SCK):
            pt = t + _NB - 1
            if pt < _SCK:
                _scat_wait(pt - _NB, pt % _NB)
                _prefetch(pt, pt % _NB)
            _process(t, t % _NB)
        return carry

    lax.fori_loop(0, _NSUP, _sup, 0)
    _drain_tail()
    plsc.subcore_barrier()

    for k in range((_NWB + _NS - 1) // _NS):
        blk = s + _NS * k

        @pl.when(blk < _NWB)
        def _wb(blk=blk):
            r0 = blk * _WB
            pltpu.sync_copy(acc_s.at[pl.ds(r0, _WB)],
                            aggp_hbm.at[c, pl.ds(r0, _WB)])


_sc_mesh = plsc.VectorSubcoreMesh(
    core_axis_name="c", subcore_axis_name="s",
    num_cores=_NC, num_subcores=_NS)

_deg_call = pl.kernel(
    _deg_body,
    out_type=jax.ShapeDtypeStruct((_NC, _N), jnp.float32),
    mesh=_sc_mesh,
    scratch_types=[
        pltpu.VMEM_SHARED((_N,), jnp.float32),
        pltpu.VMEM((_NCHUNK, _CH), jnp.int32),
        pltpu.VMEM((_EPT,), jnp.float32),
        pltpu.VMEM((_ZC,), jnp.float32),
        pltpu.SemaphoreType.DMA,
    ],
)

_agg_call = pl.kernel(
    _agg_body,
    out_type=jax.ShapeDtypeStruct((_NC, _N, _H), jnp.float32),
    mesh=_sc_mesh,
    scratch_types=[
        pltpu.VMEM_SHARED((_N, _H), jnp.float32),
        pltpu.VMEM((_SE,), jnp.int32),
        pltpu.VMEM((_SCK, _CH), jnp.int32),
        pltpu.VMEM((_SE,), jnp.float32),
        pltpu.VMEM((_CH, _H), jnp.float32),
        pltpu.VMEM((_CH, _H), jnp.float32),
        pltpu.VMEM((_CH, _H), jnp.float32),
        pltpu.VMEM((_CH, _H), jnp.float32),
        pltpu.SemaphoreType.DMA,
        pltpu.SemaphoreType.DMA,
        pltpu.SemaphoreType.DMA,
        pltpu.SemaphoreType.DMA,
        pltpu.SemaphoreType.DMA,
        pltpu.SemaphoreType.DMA,
        pltpu.SemaphoreType.DMA,
        pltpu.SemaphoreType.DMA,
    ],
)


def _prep_tc(degp_ref, node_ref, w1_ref, b1_ref, y_ref, st_ref, dis_ref, inv_ref):
    deg = degp_ref[:, 0:1] + degp_ref[:, 1:2] + 1.0
    dis = lax.rsqrt(deg)
    inv = 1.0 / deg
    xw = jnp.dot(node_ref[...], w1_ref[...], preferred_element_type=jnp.float32)
    y_ref[...] = xw * dis
    st_ref[...] = xw * inv + b1_ref[...][None, :]
    dis_ref[...] = dis
    inv_ref[...] = inv


def _combine_tc(aggp_ref, st_ref, dis_ref, inv_ref, w_ref, b_ref, y_ref, stn_ref):
    agg = aggp_ref[0] + aggp_ref[1]
    x = jnp.maximum(dis_ref[...] * agg + st_ref[...], 0.0)
    xw = jnp.dot(x, w_ref[...], preferred_element_type=jnp.float32)
    y_ref[...] = xw * dis_ref[...]
    stn_ref[...] = xw * inv_ref[...] + b_ref[...][None, :] + x


def _final_tc(aggp_ref, st_ref, dis_ref, wfc_ref, bfc_ref, out_ref):
    agg = aggp_ref[0] + aggp_ref[1]
    x = jnp.maximum(dis_ref[...] * agg + st_ref[...], 0.0)
    m = jnp.mean(x, axis=0, keepdims=True)
    out_ref[...] = (jnp.dot(m, wfc_ref[...], preferred_element_type=jnp.float32)
                    + bfc_ref[...][None, :])


def _prep_call(degp_t, node, W1, b1):
    return pl.pallas_call(
        _prep_tc,
        out_shape=(
            jax.ShapeDtypeStruct((_N, _H), jnp.float32),
            jax.ShapeDtypeStruct((_N, _H), jnp.float32),
            jax.ShapeDtypeStruct((_N, 1), jnp.float32),
            jax.ShapeDtypeStruct((_N, 1), jnp.float32),
        ),
    )(degp_t, node, W1, b1)


def _combine_call(aggp, st, dis2, inv2, W, b):
    return pl.pallas_call(
        _combine_tc,
        out_shape=(
            jax.ShapeDtypeStruct((_N, _H), jnp.float32),
            jax.ShapeDtypeStruct((_N, _H), jnp.float32),
        ),
    )(aggp, st, dis2, inv2, W, b)


def _final_call(aggp, st, dis2, Wfc, bfc):
    return pl.pallas_call(
        _final_tc,
        out_shape=jax.ShapeDtypeStruct((1, 2), jnp.float32),
    )(aggp, st, dis2, Wfc, bfc)


def kernel(node, edges, edges_attr, W1, b1, W2_0, b2_0, W2_1, b2_1, W2_2, b2_2, Wfc, bfc):
    src, dst = edges[0], edges[1]
    nt = _NC * _NS
    dst3 = dst.reshape(nt, _NCHUNK, _CH)
    dst4 = dst.reshape(nt, _NSUP, _SCK, _CH)
    degp = _deg_call(dst3, edges_attr)
    degp_t = degp.T
    y, st, dis2, inv2 = _prep_call(degp_t, node, W1, b1)
    for (W, b) in ((W2_0, b2_0), (W2_1, b2_1), (W2_2, b2_2)):
        aggp = _agg_call(y, src, dst4, edges_attr)
        y, st = _combine_call(aggp, st, dis2, inv2, W, b)
    aggp = _agg_call(y, src, dst4, edges_attr)
    return _final_call(aggp, st, dis2, Wfc, bfc)
